# baseline (speedup 1.0000x reference)
import functools

import jax
import jax.numpy as jnp
import numpy as np
from jax import lax
from jax.experimental import pallas as pl
from jax.experimental.pallas import tpu as pltpu
from jax.experimental.pallas import tpu_sc as plsc

N = 10000
E = 320000
D = 128
NT = 10112
NC, NS = 2, 16
NW = NC * NS
C = 48
CH = 212
EPW = C * CH
EPAD = NW * EPW
LANES = 16


def _gate_body(h_ref, w_ref, b_ref, o_ref):
    o_ref[...] = (
        jnp.dot(h_ref[...], w_ref[...], preferred_element_type=jnp.float32,
                precision=jax.lax.Precision.HIGHEST)
        + b_ref[...]
    )


def _gate_tc(hp, w2, b8):
    return pl.pallas_call(
        _gate_body,
        out_shape=jax.ShapeDtypeStruct((NT, 8), jnp.float32),
    )(hp, w2, b8)


def _add_body(z_ref, o_ref):
    o_ref[...] = z_ref[0] + z_ref[1]


def _add_tc(z2):
    blk = 2000
    return pl.pallas_call(
        _add_body,
        grid=(N // blk,),
        in_specs=[pl.BlockSpec((2, blk, D), lambda i: (0, i, 0))],
        out_specs=pl.BlockSpec((blk, D), lambda i: (i, 0)),
        out_shape=jax.ShapeDtypeStruct((N, D), jnp.float32),
    )(z2)


def _tanh(x):
    ax = jnp.abs(x)
    t = 1.0 - 2.0 / (jnp.exp(2.0 * ax) + 1.0)
    return jnp.where(x < 0.0, -t, t)


def _edge_body(pk_hbm, ga_hbm, gb_hbm, d_hbm, e_hbm,
               ga_v, gb_v, d_v, pk_v, e_v):
    cid = lax.axis_index("c")
    sid = lax.axis_index("s")
    wid = sid * NC + cid
    base = wid * EPW

    pltpu.sync_copy(ga_hbm, ga_v)
    pltpu.sync_copy(gb_hbm, gb_v)
    pltpu.sync_copy(d_hbm, d_v)
    pltpu.sync_copy(pk_hbm.at[pl.ds(base, EPW)], pk_v)

    def ebody(i, _):
        sl = pl.ds(i * LANES, LANES)
        p = pk_v[sl]
        s16 = p & 0xFFFF
        t16 = p >> 16
        x = plsc.load_gather(ga_v, [t16]) + plsc.load_gather(gb_v, [s16])
        e_v[sl] = (_tanh(x) * plsc.load_gather(d_v, [t16])
                   * plsc.load_gather(d_v, [s16]))
        return 0

    lax.fori_loop(0, EPW // LANES, ebody, 0, unroll=2)

    pltpu.sync_copy(e_v, e_hbm.at[pl.ds(base, EPW)])


_edge_kernel = functools.partial(
    pl.kernel,
    out_type=jax.ShapeDtypeStruct((EPAD,), jnp.float32),
    mesh=plsc.VectorSubcoreMesh(core_axis_name="c", subcore_axis_name="s",
                                num_cores=NC, num_subcores=NS),
    compiler_params=pltpu.CompilerParams(needs_layout_passes=False),
    scratch_types=[
        pltpu.VMEM((NT,), jnp.float32),
        pltpu.VMEM((NT,), jnp.float32),
        pltpu.VMEM((NT,), jnp.float32),
        pltpu.VMEM((EPW,), jnp.int32),
        pltpu.VMEM((EPW,), jnp.float32),
    ],
)(_edge_body)


def _msg_body(h_hbm, pk_hbm, e_hbm, z0_hbm,
              out_hbm,
              pk_v, sbuf, dbuf, raw, scaled, ech, z_sh,
              gsem0, gsem1, ssem0, ssem1):
    cid = lax.axis_index("c")
    sid = lax.axis_index("s")
    wid = sid * NC + cid
    base = wid * EPW

    pltpu.sync_copy(pk_hbm.at[pl.ds(base, EPW)], pk_v)

    zs = NT // NS
    pltpu.sync_copy(z0_hbm.at[pl.ds(sid * zs, zs)],
                    z_sh.at[pl.ds(sid * zs, zs)])

    plsc.subcore_barrier()

    gsems = (gsem0, gsem1)
    ssems = (ssem0, ssem1)

    def unpack_src(c, b):
        for k in range(C // LANES):
            sl = pl.ds(k * LANES, LANES)
            p = pk_v[pl.ds(c * C + k * LANES, LANES)]
            sbuf[b, sl] = p & 0xFFFF

    def unpack_dst(c, b):
        for k in range(C // LANES):
            sl = pl.ds(k * LANES, LANES)
            p = pk_v[pl.ds(c * C + k * LANES, LANES)]
            dbuf[b, sl] = p >> 16

    def start_gather(b, c):
        pltpu.async_copy(h_hbm.at[sbuf.at[b]], raw.at[b], gsems[b])
        pltpu.async_copy(e_hbm.at[pl.ds(base + c * C, C)], ech.at[b],
                         gsems[b])

    def wait_gather(b, c):
        pltpu.make_async_copy(h_hbm.at[sbuf.at[b]], raw.at[b],
                              gsems[b]).wait()
        pltpu.make_async_copy(e_hbm.at[pl.ds(base + c * C, C)], ech.at[b],
                              gsems[b]).wait()

    def start_scatter(b):
        pltpu.async_copy(scaled.at[b], z_sh.at[dbuf.at[b]], ssems[b],
                         add=True)

    def wait_scatter(b):
        pltpu.make_async_copy(scaled.at[b], z_sh.at[dbuf.at[b]],
                              ssems[b]).wait()

    for b in range(2):
        unpack_src(jnp.int32(b), b)
        start_gather(b, jnp.int32(b))

    def chunk(g, _):
        for b in range(2):
            c = g * 2 + b
            wait_gather(b, c)

            @pl.when(g >= 1)
            def _():
                wait_scatter(b)

            unpack_dst(c, b)

            def qbody(q, _):
                e16 = ech[b, pl.ds(q * LANES, LANES)]
                for k in range(LANES):
                    es = jnp.full((LANES,), e16[k], jnp.float32)
                    r = q * LANES + k
                    for j in range(D // LANES):
                        fl = pl.ds(j * LANES, LANES)
                        scaled[b, r, fl] = raw[b, r, fl] * es
                return 0

            lax.fori_loop(0, C // LANES, qbody, 0)

            start_scatter(b)

            @pl.when(c + 2 < CH)
            def _():
                unpack_src(c + 2, b)
                start_gather(b, c + 2)
        return 0

    lax.fori_loop(0, CH // 2, chunk, 0)

    for b in range(2):
        wait_scatter(b)

    plsc.subcore_barrier()

    pltpu.sync_copy(z_sh.at[pl.ds(sid * zs, zs)],
                    out_hbm.at[cid, pl.ds(sid * zs, zs)])


_msg_kernel = functools.partial(
    pl.kernel,
    out_type=jax.ShapeDtypeStruct((NC, NT, D), jnp.float32),
    mesh=plsc.VectorSubcoreMesh(core_axis_name="c", subcore_axis_name="s",
                                num_cores=NC, num_subcores=NS),
    compiler_params=pltpu.CompilerParams(needs_layout_passes=False),
    scratch_types=[
        pltpu.VMEM((EPW,), jnp.int32),
        pltpu.VMEM((2, C), jnp.int32),
        pltpu.VMEM((2, C), jnp.int32),
        pltpu.VMEM((2, C, D), jnp.float32),
        pltpu.VMEM((2, C, D), jnp.float32),
        pltpu.VMEM((2, C), jnp.float32),
        pltpu.VMEM_SHARED((NT, D), jnp.float32),
        pltpu.SemaphoreType.DMA,
        pltpu.SemaphoreType.DMA,
        pltpu.SemaphoreType.DMA,
        pltpu.SemaphoreType.DMA,
    ],
)(_msg_body)


def kernel(h, edge_index, d, gate_w, gate_b):
    h = h.astype(jnp.float32)
    src = edge_index[0].astype(jnp.int32)
    dst = edge_index[1].astype(jnp.int32)
    pad = EPAD - E
    packed = (dst << 16) | src
    pk = jnp.concatenate([packed, jnp.full((pad,), N << 16, jnp.int32)])
    d_p = jnp.pad(d.astype(jnp.float32), (0, NT - N))
    hp = jnp.pad(h, ((0, NT - N), (0, 0)))

    w2 = jnp.zeros((D, 8), jnp.float32)
    w2 = w2.at[:, 0].set(gate_w[0, :D].astype(jnp.float32))
    w2 = w2.at[:, 1].set(gate_w[0, D:].astype(jnp.float32))
    b8 = jnp.zeros((1, 8), jnp.float32).at[0, 0].set(gate_b[0].astype(jnp.float32))

    gg = _gate_tc(hp, w2, b8)
    ga = gg[:, 0]
    gb = gg[:, 1]

    e_all = _edge_kernel(pk, ga, gb, d_p)
    z0 = jnp.zeros((NT, D), jnp.float32)
    z2 = _msg_kernel(hp, pk, e_all, z0)
    return _add_tc(z2)


# speedup vs baseline: 1.0251x; 1.0251x over previous
import functools

import jax
import jax.numpy as jnp
import numpy as np
from jax import lax
from jax.experimental import pallas as pl
from jax.experimental.pallas import tpu as pltpu
from jax.experimental.pallas import tpu_sc as plsc

N = 10000
E = 320000
D = 128
NT = 10112
NP = 10016
NC, NS = 2, 16
NW = NC * NS
C = 48
CH = 212
EPW = C * CH
EPAD = NW * EPW
LANES = 16


def _gate_body(h_ref, w_ref, b_ref, o_ref):
    o_ref[...] = (
        jnp.dot(h_ref[...], w_ref[...], preferred_element_type=jnp.float32,
                precision=jax.lax.Precision.HIGHEST)
        + b_ref[...]
    )


def _gate_tc(h, w2, b8):
    return pl.pallas_call(
        _gate_body,
        out_shape=jax.ShapeDtypeStruct((N, 8), jnp.float32),
    )(h, w2, b8)


def _add_body(z_ref, o_ref):
    o_ref[...] = z_ref[0] + z_ref[1]


def _add_tc(z2):
    blk = 2000
    return pl.pallas_call(
        _add_body,
        grid=(N // blk,),
        in_specs=[pl.BlockSpec((2, blk, D), lambda i: (0, i, 0))],
        out_specs=pl.BlockSpec((blk, D), lambda i: (i, 0)),
        out_shape=jax.ShapeDtypeStruct((N, D), jnp.float32),
    )(z2)


def _tanh(x):
    ax = jnp.abs(x)
    t = 1.0 - 2.0 / (jnp.exp(2.0 * ax) + 1.0)
    return jnp.where(x < 0.0, -t, t)


def _edge_body(pk_hbm, ga_hbm, gb_hbm, d_hbm, e_hbm,
               ga_v, gb_v, d_v, pk_v, e_v):
    cid = lax.axis_index("c")
    sid = lax.axis_index("s")
    wid = sid * NC + cid
    base = wid * EPW

    pltpu.sync_copy(ga_hbm, ga_v)
    pltpu.sync_copy(gb_hbm, gb_v)
    pltpu.sync_copy(d_hbm, d_v)
    pltpu.sync_copy(pk_hbm.at[pl.ds(base, EPW)], pk_v)

    def ebody(i, _):
        sl = pl.ds(i * LANES, LANES)
        p = pk_v[sl]
        s16 = p & 0xFFFF
        t16 = p >> 16
        x = plsc.load_gather(ga_v, [t16]) + plsc.load_gather(gb_v, [s16])
        e_v[sl] = (_tanh(x) * plsc.load_gather(d_v, [t16])
                   * plsc.load_gather(d_v, [s16]))
        return 0

    lax.fori_loop(0, EPW // LANES, ebody, 0, unroll=2)

    pltpu.sync_copy(e_v, e_hbm.at[pl.ds(base, EPW)])


_edge_kernel = functools.partial(
    pl.kernel,
    out_type=jax.ShapeDtypeStruct((EPAD,), jnp.float32),
    mesh=plsc.VectorSubcoreMesh(core_axis_name="c", subcore_axis_name="s",
                                num_cores=NC, num_subcores=NS),
    compiler_params=pltpu.CompilerParams(needs_layout_passes=False),
    scratch_types=[
        pltpu.VMEM((NP,), jnp.float32),
        pltpu.VMEM((NP,), jnp.float32),
        pltpu.VMEM((NP,), jnp.float32),
        pltpu.VMEM((EPW,), jnp.int32),
        pltpu.VMEM((EPW,), jnp.float32),
    ],
)(_edge_body)


def _msg_body(h_hbm, pk_hbm, e_hbm,
              out_hbm,
              pk_v, sbuf, dbuf, raw, scaled, ech, z_sh,
              gsem0, gsem1, ssem0, ssem1):
    cid = lax.axis_index("c")
    sid = lax.axis_index("s")
    wid = sid * NC + cid
    base = wid * EPW

    pltpu.sync_copy(pk_hbm.at[pl.ds(base, EPW)], pk_v)

    zvec = jnp.zeros((LANES,), jnp.float32)

    def zbody(r, _):
        for j in range(D // LANES):
            scaled[0, r, pl.ds(j * LANES, LANES)] = zvec
        return 0

    lax.fori_loop(0, C, zbody, 0)
    zs = NT // NS
    for k in range(zs // C):
        pltpu.sync_copy(scaled.at[0],
                        z_sh.at[pl.ds(sid * zs + k * C, C)])
    rem = zs - (zs // C) * C
    if rem:
        pltpu.sync_copy(scaled.at[0, pl.ds(0, rem)],
                        z_sh.at[pl.ds(sid * zs + (zs // C) * C, rem)])

    plsc.subcore_barrier()

    gsems = (gsem0, gsem1)
    ssems = (ssem0, ssem1)

    def unpack_src(c, b):
        for k in range(C // LANES):
            sl = pl.ds(k * LANES, LANES)
            p = pk_v[pl.ds(c * C + k * LANES, LANES)]
            sbuf[b, sl] = p & 0xFFFF

    def unpack_dst(c, b):
        for k in range(C // LANES):
            sl = pl.ds(k * LANES, LANES)
            p = pk_v[pl.ds(c * C + k * LANES, LANES)]
            dbuf[b, sl] = p >> 16

    def start_gather(b, c):
        pltpu.async_copy(h_hbm.at[sbuf.at[b]], raw.at[b], gsems[b])
        pltpu.async_copy(e_hbm.at[pl.ds(base + c * C, C)], ech.at[b],
                         gsems[b])

    def wait_gather(b, c):
        pltpu.make_async_copy(h_hbm.at[sbuf.at[b]], raw.at[b],
                              gsems[b]).wait()
        pltpu.make_async_copy(e_hbm.at[pl.ds(base + c * C, C)], ech.at[b],
                              gsems[b]).wait()

    def start_scatter(b):
        pltpu.async_copy(scaled.at[b], z_sh.at[dbuf.at[b]], ssems[b],
                         add=True)

    def wait_scatter(b):
        pltpu.make_async_copy(scaled.at[b], z_sh.at[dbuf.at[b]],
                              ssems[b]).wait()

    for b in range(2):
        unpack_src(jnp.int32(b), b)
        start_gather(b, jnp.int32(b))

    def chunk(g, _):
        for b in range(2):
            c = g * 2 + b
            wait_gather(b, c)

            @pl.when(g >= 1)
            def _():
                wait_scatter(b)

            unpack_dst(c, b)

            def qbody(q, _):
                e16 = ech[b, pl.ds(q * LANES, LANES)]
                for k in range(LANES):
                    es = jnp.full((LANES,), e16[k], jnp.float32)
                    r = q * LANES + k
                    for j in range(D // LANES):
                        fl = pl.ds(j * LANES, LANES)
                        scaled[b, r, fl] = raw[b, r, fl] * es
                return 0

            lax.fori_loop(0, C // LANES, qbody, 0)

            start_scatter(b)

            @pl.when(c + 2 < CH)
            def _():
                unpack_src(c + 2, b)
                start_gather(b, c + 2)
        return 0

    lax.fori_loop(0, CH // 2, chunk, 0)

    for b in range(2):
        wait_scatter(b)

    plsc.subcore_barrier()

    pltpu.sync_copy(z_sh.at[pl.ds(sid * zs, zs)],
                    out_hbm.at[cid, pl.ds(sid * zs, zs)])


_msg_kernel = functools.partial(
    pl.kernel,
    out_type=jax.ShapeDtypeStruct((NC, NT, D), jnp.float32),
    mesh=plsc.VectorSubcoreMesh(core_axis_name="c", subcore_axis_name="s",
                                num_cores=NC, num_subcores=NS),
    compiler_params=pltpu.CompilerParams(needs_layout_passes=False),
    scratch_types=[
        pltpu.VMEM((EPW,), jnp.int32),
        pltpu.VMEM((2, C), jnp.int32),
        pltpu.VMEM((2, C), jnp.int32),
        pltpu.VMEM((2, C, D), jnp.float32),
        pltpu.VMEM((2, C, D), jnp.float32),
        pltpu.VMEM((2, C), jnp.float32),
        pltpu.VMEM_SHARED((NT, D), jnp.float32),
        pltpu.SemaphoreType.DMA,
        pltpu.SemaphoreType.DMA,
        pltpu.SemaphoreType.DMA,
        pltpu.SemaphoreType.DMA,
    ],
)(_msg_body)


def kernel(h, edge_index, d, gate_w, gate_b):
    h = h.astype(jnp.float32)
    src = edge_index[0].astype(jnp.int32)
    dst = edge_index[1].astype(jnp.int32)
    pad = EPAD - E
    packed = (dst << 16) | src
    pk = jnp.concatenate([packed, jnp.full((pad,), N << 16, jnp.int32)])
    d_p = jnp.pad(d.astype(jnp.float32), (0, NP - N))

    w2 = jnp.zeros((D, 8), jnp.float32)
    w2 = w2.at[:, 0].set(gate_w[0, :D].astype(jnp.float32))
    w2 = w2.at[:, 1].set(gate_w[0, D:].astype(jnp.float32))
    b8 = jnp.zeros((1, 8), jnp.float32).at[0, 0].set(gate_b[0].astype(jnp.float32))

    gg = _gate_tc(h, w2, b8)
    gp = jnp.pad(gg, ((0, NP - N), (0, 0)))
    ga = gp[:, 0]
    gb = gp[:, 1]

    e_all = _edge_kernel(pk, ga, gb, d_p)
    z2 = _msg_kernel(h, pk, e_all)
    return _add_tc(z2)


# speedup vs baseline: 1.0497x; 1.0240x over previous
import functools

import jax
import jax.numpy as jnp
import numpy as np
from jax import lax
from jax.experimental import pallas as pl
from jax.experimental.pallas import tpu as pltpu
from jax.experimental.pallas import tpu_sc as plsc

N = 10000
E = 320000
D = 128
NT = 10112
NP = 10016
NC, NS = 2, 16
NW = NC * NS
C = 48
CH = 212
EPW = C * CH
EPAD = NW * EPW
LANES = 16


def _gate_body(h_ref, w_ref, b_ref, o_ref):
    o_ref[...] = (
        jnp.dot(h_ref[...], w_ref[...], preferred_element_type=jnp.float32,
                precision=jax.lax.Precision.HIGHEST)
        + b_ref[...]
    )


def _gate_tc(h, w2, b8):
    return pl.pallas_call(
        _gate_body,
        out_shape=jax.ShapeDtypeStruct((N, 8), jnp.float32),
    )(h, w2, b8)


def _add_body(z_ref, o_ref):
    o_ref[...] = z_ref[0] + z_ref[1]


def _add_tc(z2):
    blk = 2000
    return pl.pallas_call(
        _add_body,
        grid=(N // blk,),
        in_specs=[pl.BlockSpec((2, blk, D), lambda i: (0, i, 0))],
        out_specs=pl.BlockSpec((blk, D), lambda i: (i, 0)),
        out_shape=jax.ShapeDtypeStruct((N, D), jnp.float32),
    )(z2)


def _tanh(x):
    ax = jnp.abs(x)
    t = 1.0 - 2.0 / (jnp.exp(2.0 * ax) + 1.0)
    return jnp.where(x < 0.0, -t, t)


def _edge_body(pk_hbm, ga_hbm, gb_hbm, d_hbm, e_hbm,
               ga_v, gb_v, d_v, pk_v, e_v):
    cid = lax.axis_index("c")
    sid = lax.axis_index("s")
    wid = sid * NC + cid
    base = wid * EPW

    pltpu.sync_copy(ga_hbm, ga_v)
    pltpu.sync_copy(gb_hbm, gb_v)
    pltpu.sync_copy(d_hbm, d_v)
    pltpu.sync_copy(pk_hbm.at[pl.ds(base, EPW)], pk_v)

    def ebody(i, _):
        sl = pl.ds(i * LANES, LANES)
        p = pk_v[sl]
        s16 = p & 0xFFFF
        t16 = p >> 16
        x = plsc.load_gather(ga_v, [t16]) + plsc.load_gather(gb_v, [s16])
        e_v[sl] = (_tanh(x) * plsc.load_gather(d_v, [t16])
                   * plsc.load_gather(d_v, [s16]))
        return 0

    lax.fori_loop(0, EPW // LANES, ebody, 0, unroll=2)

    pltpu.sync_copy(e_v, e_hbm.at[pl.ds(base, EPW)])


_edge_kernel = functools.partial(
    pl.kernel,
    out_type=jax.ShapeDtypeStruct((EPAD,), jnp.float32),
    mesh=plsc.VectorSubcoreMesh(core_axis_name="c", subcore_axis_name="s",
                                num_cores=NC, num_subcores=NS),
    compiler_params=pltpu.CompilerParams(needs_layout_passes=False),
    scratch_types=[
        pltpu.VMEM((NP,), jnp.float32),
        pltpu.VMEM((NP,), jnp.float32),
        pltpu.VMEM((NP,), jnp.float32),
        pltpu.VMEM((EPW,), jnp.int32),
        pltpu.VMEM((EPW,), jnp.float32),
    ],
)(_edge_body)


def _msg_body(h_hbm, pk_hbm, e_hbm,
              out_hbm,
              pk_v, sbuf, dbuf, raw, scaled, ech, z_sh,
              gsem0, gsem1, ssem0, ssem1):
    cid = lax.axis_index("c")
    sid = lax.axis_index("s")
    wid = sid * NC + cid
    base = wid * EPW

    pltpu.sync_copy(pk_hbm.at[pl.ds(base, EPW)], pk_v)

    zvec = jnp.zeros((LANES,), jnp.float32)

    def zbody(r, _):
        for j in range(D // LANES):
            scaled[0, r, pl.ds(j * LANES, LANES)] = zvec
        return 0

    lax.fori_loop(0, C, zbody, 0)
    zs = NT // NS
    for k in range(zs // C):
        pltpu.sync_copy(scaled.at[0],
                        z_sh.at[pl.ds(sid * zs + k * C, C)])
    rem = zs - (zs // C) * C
    if rem:
        pltpu.sync_copy(scaled.at[0, pl.ds(0, rem)],
                        z_sh.at[pl.ds(sid * zs + (zs // C) * C, rem)])

    plsc.subcore_barrier()

    gsems = (gsem0, gsem1)
    ssems = (ssem0, ssem1)

    def unpack_src(c, b):
        for k in range(C // LANES):
            sl = pl.ds(k * LANES, LANES)
            p = pk_v[pl.ds(c * C + k * LANES, LANES)]
            sbuf[b, sl] = p & 0xFFFF

    def unpack_dst(c, b):
        for k in range(C // LANES):
            sl = pl.ds(k * LANES, LANES)
            p = pk_v[pl.ds(c * C + k * LANES, LANES)]
            dbuf[b, sl] = p >> 16

    def start_gather(b, c):
        pltpu.async_copy(h_hbm.at[sbuf.at[b]], raw.at[b], gsems[b])
        pltpu.async_copy(e_hbm.at[pl.ds(base + c * C, C)], ech.at[b],
                         gsems[b])

    def wait_gather(b, c):
        pltpu.make_async_copy(h_hbm.at[sbuf.at[b]], raw.at[b],
                              gsems[b]).wait()
        pltpu.make_async_copy(e_hbm.at[pl.ds(base + c * C, C)], ech.at[b],
                              gsems[b]).wait()

    def start_scatter(b):
        pltpu.async_copy(scaled.at[b], z_sh.at[dbuf.at[b]], ssems[b],
                         add=True)

    def wait_scatter(b):
        pltpu.make_async_copy(scaled.at[b], z_sh.at[dbuf.at[b]],
                              ssems[b]).wait()

    for b in range(2):
        unpack_src(jnp.int32(b), b)
        start_gather(b, jnp.int32(b))

    def chunk(g, _):
        for b in range(2):
            c = g * 2 + b
            wait_gather(b, c)

            @pl.when(g >= 1)
            def _():
                wait_scatter(b)

            unpack_dst(c, b)

            for q in range(C // LANES):
                e16 = ech[b, pl.ds(q * LANES, LANES)]
                for k in range(LANES):
                    es = jnp.full((LANES,), e16[k], jnp.float32)
                    r = q * LANES + k
                    for j in range(D // LANES):
                        fl = pl.ds(j * LANES, LANES)
                        scaled[b, r, fl] = raw[b, r, fl] * es

            start_scatter(b)

            @pl.when(c + 2 < CH)
            def _():
                unpack_src(c + 2, b)
                start_gather(b, c + 2)
        return 0

    lax.fori_loop(0, CH // 2, chunk, 0)

    for b in range(2):
        wait_scatter(b)

    plsc.subcore_barrier()

    pltpu.sync_copy(z_sh.at[pl.ds(sid * zs, zs)],
                    out_hbm.at[cid, pl.ds(sid * zs, zs)])


_msg_kernel = functools.partial(
    pl.kernel,
    out_type=jax.ShapeDtypeStruct((NC, NT, D), jnp.float32),
    mesh=plsc.VectorSubcoreMesh(core_axis_name="c", subcore_axis_name="s",
                                num_cores=NC, num_subcores=NS),
    compiler_params=pltpu.CompilerParams(needs_layout_passes=False),
    scratch_types=[
        pltpu.VMEM((EPW,), jnp.int32),
        pltpu.VMEM((2, C), jnp.int32),
        pltpu.VMEM((2, C), jnp.int32),
        pltpu.VMEM((2, C, D), jnp.float32),
        pltpu.VMEM((2, C, D), jnp.float32),
        pltpu.VMEM((2, C), jnp.float32),
        pltpu.VMEM_SHARED((NT, D), jnp.float32),
        pltpu.SemaphoreType.DMA,
        pltpu.SemaphoreType.DMA,
        pltpu.SemaphoreType.DMA,
        pltpu.SemaphoreType.DMA,
    ],
)(_msg_body)


def kernel(h, edge_index, d, gate_w, gate_b):
    h = h.astype(jnp.float32)
    src = edge_index[0].astype(jnp.int32)
    dst = edge_index[1].astype(jnp.int32)
    pad = EPAD - E
    packed = (dst << 16) | src
    pk = jnp.concatenate([packed, jnp.full((pad,), N << 16, jnp.int32)])
    d_p = jnp.pad(d.astype(jnp.float32), (0, NP - N))

    w2 = jnp.zeros((D, 8), jnp.float32)
    w2 = w2.at[:, 0].set(gate_w[0, :D].astype(jnp.float32))
    w2 = w2.at[:, 1].set(gate_w[0, D:].astype(jnp.float32))
    b8 = jnp.zeros((1, 8), jnp.float32).at[0, 0].set(gate_b[0].astype(jnp.float32))

    gg = _gate_tc(h, w2, b8)
    gp = jnp.pad(gg, ((0, NP - N), (0, 0)))
    ga = gp[:, 0]
    gb = gp[:, 1]

    e_all = _edge_kernel(pk, ga, gb, d_p)
    z2 = _msg_kernel(h, pk, e_all)
    return _add_tc(z2)
